# Initial kernel scaffold; baseline (speedup 1.0000x reference)
import jax
import jax.numpy as jnp
from jax.experimental import pallas as pl


def kernel(xyz, xyz_batch_cnt, new_xyz, rois, features, W0_0, g0_0, b0_0, W0_1, g0_1, b0_1, W1_0, g1_0, b1_0, W1_1, g1_1, b1_1):
    raise NotImplementedError("write your pallas kernel here")


# speedup vs baseline: 12.1804x; 12.1804x over previous
import functools

import jax
import jax.numpy as jnp
from jax import lax
from jax.experimental import pallas as pl
from jax.experimental.pallas import tpu as pltpu
from jax.experimental.pallas import tpu_sc as plsc

B = 2
NPER = 2048
NQ = 6912
R2 = (0.64, 2.56)
NS = (16, 32)
DPAD = 128
EPS = 1e-5
IMAX = 0x7FFFFFFF
LOWMASK = 2047

NW = 32
GCH = 384


def _select_body(qt_ref, xt_ref, outA_ref, outB_ref, key_ref, m_ref, *, qt):
    b = pl.program_id(0)
    acc = jnp.zeros((qt, NPER), jnp.float32)
    for dim in range(3):
        qd = qt_ref[0, dim, :][:, None]
        xd = xt_ref[0, dim, :][None, :]
        diff = qd - xd
        acc = acc + diff * diff
    iota = jax.lax.broadcasted_iota(jnp.int32, (qt, NPER), 1)
    keys = jax.lax.bitcast_convert_type(acc, jnp.int32)
    keys = jnp.bitwise_and(keys, jnp.int32(~LOWMASK))
    keys = jnp.bitwise_or(keys, iota)
    key_ref[...] = keys

    far = jnp.float32(1e6) + acc
    m2 = jnp.min(far, axis=1)
    am2 = jnp.min(jnp.where(far == m2[:, None], iota, jnp.int32(NPER)), axis=1)
    fill0 = am2[None, :] + b * NPER

    def body(k, _):
        kcur = key_ref[...]
        m = jnp.min(kcur, axis=1)
        key_ref[...] = jnp.where(kcur == m[:, None], jnp.int32(IMAX), kcur)
        m_ref[pl.ds(k, 1), :] = m[None, :]
        return 0

    jax.lax.fori_loop(0, 32, body, 0)

    M = m_ref[...]
    gidx = jnp.bitwise_and(M, jnp.int32(LOWMASK)) + b * NPER
    d2t = jax.lax.bitcast_convert_type(jnp.bitwise_and(M, jnp.int32(~LOWMASK)),
                                       jnp.float32)
    vA = d2t[:NS[0]] <= R2[0]
    fillA = jnp.where(vA[0:1], gidx[0:1], fill0)
    outA_ref[0] = jnp.where(vA, gidx[:NS[0]], fillA)
    vB = d2t[:NS[1]] <= R2[1]
    fillB = jnp.where(vB[0:1], gidx[0:1], fill0)
    outB_ref[0] = jnp.where(vB, gidx[:NS[1]], fillB)


def _select(q_t, x_t, qt=128):
    nsteps = NQ // qt
    kfn = pl.pallas_call(
        functools.partial(_select_body, qt=qt),
        grid=(B, nsteps),
        in_specs=[
            pl.BlockSpec((1, 3, qt), lambda b, i: (b, 0, i)),
            pl.BlockSpec((1, 3, NPER), lambda b, i: (b, 0, 0)),
        ],
        out_specs=[
            pl.BlockSpec((1, NS[0], qt), lambda b, i: (b, 0, i)),
            pl.BlockSpec((1, NS[1], qt), lambda b, i: (b, 0, i)),
        ],
        out_shape=[
            jax.ShapeDtypeStruct((B, NS[0], NQ), jnp.int32),
            jax.ShapeDtypeStruct((B, NS[1], NQ), jnp.int32),
        ],
        scratch_shapes=[
            pltpu.VMEM((qt, NPER), jnp.int32),
            pltpu.VMEM((32, qt), jnp.int32),
        ],
    )
    return kfn(q_t, x_t)


def _sc_gather(table, idx2d, nrows):
    b_per_w = nrows // NW
    nch = b_per_w // GCH
    blk_per_w = b_per_w // 128
    mesh = plsc.VectorSubcoreMesh(core_axis_name="c", subcore_axis_name="s")

    @functools.partial(
        pl.kernel, mesh=mesh,
        out_type=jax.ShapeDtypeStruct((nrows, DPAD), jnp.float32),
        scratch_types=[
            pltpu.VMEM((blk_per_w, 128), jnp.int32),
            pltpu.VMEM((GCH, DPAD), jnp.float32),
            pltpu.SemaphoreType.DMA,
        ],
    )
    def k(table_hbm, idx_hbm, out_hbm, idx_v, rows_v, sem):
        wid = lax.axis_index("s") * 2 + lax.axis_index("c")
        pltpu.sync_copy(idx_hbm.at[wid], idx_v)

        def body(i, _):
            cps = [
                pltpu.async_copy(table_hbm.at[idx_v.at[i * 3 + j]],
                                 rows_v.at[pl.ds(j * 128, 128)], sem)
                for j in range(3)
            ]
            for cp in cps:
                cp.wait()
            pltpu.sync_copy(
                rows_v, out_hbm.at[pl.ds((wid * blk_per_w + i * 3) * 128, GCH)])
            return 0

        lax.fori_loop(0, nch, body, 0)

    return k(table, idx2d)


def _stats1_body(g_ref, q_ref, w1_ref, out_ref, *, qt, ns, d1):
    i = pl.program_id(0)
    G = g_ref[...]
    W = w1_ref[...]
    y = jnp.dot(G, W, preferred_element_type=jnp.float32)
    y = y.reshape(qt, ns, d1)
    qw = jnp.dot(q_ref[...], W[0:3, :], preferred_element_type=jnp.float32)
    y = y - qw[:, None, :]
    s = jnp.sum(y, axis=(0, 1))
    ss = jnp.sum(y * y, axis=(0, 1))

    @pl.when(i == 0)
    def _():
        out_ref[...] = jnp.zeros_like(out_ref)

    out_ref[0, :] += s
    out_ref[1, :] += ss


def _layer1(g_ref, q_ref, w1_ref, a1_ref, b1_ref, qt, ns, d1):
    G = g_ref[...]
    W = w1_ref[...]
    y = jnp.dot(G, W, preferred_element_type=jnp.float32)
    y = y.reshape(qt, ns, d1)
    qw = jnp.dot(q_ref[...], W[0:3, :], preferred_element_type=jnp.float32)
    y = y - qw[:, None, :]
    return jnp.maximum(y * a1_ref[0][None, None, :] + b1_ref[0][None, None, :],
                       0.0)


def _stats2_body(g_ref, q_ref, w1_ref, a1_ref, b1_ref, w2_ref, out_ref, *,
                 qt, ns, d1):
    i = pl.program_id(0)
    a1 = _layer1(g_ref, q_ref, w1_ref, a1_ref, b1_ref, qt, ns, d1)
    y2 = jnp.dot(a1.reshape(qt * ns, d1), w2_ref[...],
                 preferred_element_type=jnp.float32)
    s = jnp.sum(y2, axis=0)
    ss = jnp.sum(y2 * y2, axis=0)

    @pl.when(i == 0)
    def _():
        out_ref[...] = jnp.zeros_like(out_ref)

    out_ref[0, :] += s
    out_ref[1, :] += ss


def _final_body(g_ref, q_ref, w1_ref, a1_ref, b1_ref, w2_ref, a2_ref, b2_ref,
                out_ref, *, qt, ns, d1):
    a1 = _layer1(g_ref, q_ref, w1_ref, a1_ref, b1_ref, qt, ns, d1)
    y2 = jnp.dot(a1.reshape(qt * ns, d1), w2_ref[...],
                 preferred_element_type=jnp.float32)
    a2 = jnp.maximum(y2 * a2_ref[0][None, :] + b2_ref[0][None, :], 0.0)
    out_ref[...] = jnp.max(a2.reshape(qt, ns, 64), axis=1)


def _mlp_scale(G, q_flat, W1p, g1, b1, W2, g2, b2, ns, d1, qt=128):
    n = G.shape[0]
    nq = n // ns
    grid = (nq // qt,)
    g_spec = pl.BlockSpec((qt * ns, DPAD), lambda i: (i, 0))
    q_spec = pl.BlockSpec((qt, 3), lambda i: (i, 0))
    w1_spec = pl.BlockSpec((DPAD, d1), lambda i: (0, 0))
    vec1 = pl.BlockSpec((1, d1), lambda i: (0, 0))
    w2_spec = pl.BlockSpec((d1, 64), lambda i: (0, 0))
    vec2 = pl.BlockSpec((1, 64), lambda i: (0, 0))

    st1 = pl.pallas_call(
        functools.partial(_stats1_body, qt=qt, ns=ns, d1=d1),
        grid=grid,
        in_specs=[g_spec, q_spec, w1_spec],
        out_specs=pl.BlockSpec((2, d1), lambda i: (0, 0)),
        out_shape=jax.ShapeDtypeStruct((2, d1), jnp.float32),
    )(G, q_flat, W1p)
    mu1 = st1[0] / n
    var1 = st1[1] / n - mu1 * mu1
    al1 = (g1 / jnp.sqrt(var1 + EPS))[None, :]
    be1 = (b1 - mu1 * al1[0])[None, :]

    st2 = pl.pallas_call(
        functools.partial(_stats2_body, qt=qt, ns=ns, d1=d1),
        grid=grid,
        in_specs=[g_spec, q_spec, w1_spec, vec1, vec1, w2_spec],
        out_specs=pl.BlockSpec((2, 64), lambda i: (0, 0)),
        out_shape=jax.ShapeDtypeStruct((2, 64), jnp.float32),
    )(G, q_flat, W1p, al1, be1, W2)
    mu2 = st2[0] / n
    var2 = st2[1] / n - mu2 * mu2
    al2 = (g2 / jnp.sqrt(var2 + EPS))[None, :]
    be2 = (b2 - mu2 * al2[0])[None, :]

    out = pl.pallas_call(
        functools.partial(_final_body, qt=qt, ns=ns, d1=d1),
        grid=grid,
        in_specs=[g_spec, q_spec, w1_spec, vec1, vec1, w2_spec, vec2, vec2],
        out_specs=pl.BlockSpec((qt, 64), lambda i: (i, 0)),
        out_shape=jax.ShapeDtypeStruct((nq, 64), jnp.float32),
    )(G, q_flat, W1p, al1, be1, W2, al2, be2)
    return out


def kernel(xyz, xyz_batch_cnt, new_xyz, rois, features,
           W0_0, g0_0, b0_0, W0_1, g0_1, b0_1,
           W1_0, g1_0, b1_0, W1_1, g1_1, b1_1):
    x_t = xyz.reshape(B, NPER, 3).transpose(0, 2, 1)
    q_flat = new_xyz.reshape(B * NQ, 3)
    q_t = new_xyz.reshape(B, NQ, 3).transpose(0, 2, 1)

    idxA3, idxB3 = _select(q_t, x_t)
    idxA = idxA3.transpose(0, 2, 1).reshape(NW, -1, 128)
    idxB = idxB3.transpose(0, 2, 1).reshape(NW, -1, 128)

    table = jnp.concatenate(
        [xyz, features, jnp.zeros((B * NPER, DPAD - 35), jnp.float32)], axis=1)

    GA = _sc_gather(table, idxA, B * NQ * NS[0])
    GB = _sc_gather(table, idxB, B * NQ * NS[1])

    W0p = jnp.concatenate(
        [W0_0, jnp.zeros((DPAD - 35, W0_0.shape[1]), jnp.float32)], axis=0)
    W1p = jnp.concatenate(
        [W1_0, jnp.zeros((DPAD - 35, W1_0.shape[1]), jnp.float32)], axis=0)

    outA = _mlp_scale(GA, q_flat, W0p, g0_0, b0_0, W0_1, g0_1, b0_1, NS[0], 32)
    outB = _mlp_scale(GB, q_flat, W1p, g1_0, b1_0, W1_1, g1_1, b1_1, NS[1], 64)
    return jnp.concatenate([outA, outB], axis=1)
